# Initial kernel scaffold; baseline (speedup 1.0000x reference)
import functools

import jax
import jax.numpy as jnp
from jax import lax
from jax.experimental import pallas as pl
from jax.experimental.pallas import tpu as pltpu
from jax.experimental.pallas import tpu_sc as plsc

NC = 2
NS = 16
NW = NC * NS
R = 64
DK = 8


def _seg_pool_kernel(M, D, S, NP):
    mesh = plsc.VectorSubcoreMesh(core_axis_name="c", subcore_axis_name="s")
    S1 = S + 1

    @functools.partial(
        pl.kernel,
        out_type=(
            jax.ShapeDtypeStruct((NP, D), jnp.float32),
            jax.ShapeDtypeStruct((NP, D), jnp.float32),
        ),
        mesh=mesh,
        scratch_types=(
            pltpu.VMEM((40,), jnp.int32),
            pltpu.VMEM((R,), jnp.int32),
            pltpu.VMEM((R, D), jnp.float32),
            pltpu.VMEM((S1, D), jnp.float32),
            pltpu.VMEM((S1, D), jnp.float32),
            pltpu.SMEM((S1,), jnp.int32),
        ),
    )
    def seg_pool(seg_hbm, lanes_hbm, starts_hbm, omax_hbm, omean_hbm,
                 starts_v, seg_buf, rows_buf, acc_max, acc_sum, counts):
        wid = lax.axis_index("s") * NC + lax.axis_index("c")
        base_seg = wid * S

        pltpu.sync_copy(starts_hbm, starts_v)
        start = starts_v[wid]
        end = starts_v[wid + 1]
        astart = start - lax.rem(start, 8)
        nt = lax.div(end - astart + (R - 1), R)

        neg_inf = jnp.full((16,), -jnp.inf, dtype=jnp.float32)
        zeros = jnp.zeros((16,), dtype=jnp.float32)

        def init_body(i, _):
            for k in range(DK):
                sl = pl.ds(k * 16, 16)
                acc_max[i, sl] = neg_inf
                acc_sum[i, sl] = zeros
            counts[i] = 0
            return 0

        lax.fori_loop(0, S1, init_body, 0)

        def tile_body(t, _):
            q_t = astart + t * R
            q = jnp.minimum(q_t, M - R)
            pltpu.sync_copy(seg_hbm.at[pl.ds(q, R)], seg_buf)
            pltpu.sync_copy(lanes_hbm.at[pl.ds(q, R), :], rows_buf)
            i_lo = q_t - q
            i_hi = jnp.minimum(end - q, R)

            def row_body(i, _):
                s = seg_buf[i]
                loc = s - base_seg
                loc = jnp.where(loc < 0, S, jnp.minimum(loc, S))
                for k in range(DK):
                    sl = pl.ds(k * 16, 16)
                    r = rows_buf[i, sl]
                    acc_max[loc, sl] = jnp.maximum(acc_max[loc, sl], r)
                    acc_sum[loc, sl] = acc_sum[loc, sl] + r
                counts[loc] = counts[loc] + 1
                return 0

            lax.fori_loop(i_lo, i_hi, row_body, 0)
            return 0

        lax.fori_loop(0, nt, tile_body, 0)

        def fin_body(i, _):
            c = counts[i]
            cf = jnp.broadcast_to(c, (16,)).astype(jnp.float32)
            inv = 1.0 / jnp.maximum(cf, 1.0)
            nz = cf > 0.0
            for k in range(DK):
                sl = pl.ds(k * 16, 16)
                acc_max[i, sl] = jnp.where(nz, acc_max[i, sl], 0.0)
                acc_sum[i, sl] = acc_sum[i, sl] * inv
            return 0

        lax.fori_loop(0, S, fin_body, 0)

        pltpu.sync_copy(acc_max.at[pl.ds(0, S), :],
                        omax_hbm.at[pl.ds(base_seg, S), :])
        pltpu.sync_copy(acc_sum.at[pl.ds(0, S), :],
                        omean_hbm.at[pl.ds(base_seg, S), :])

    return seg_pool


def _linear_relu_kernel(pmax_ref, pmean_ref, wmax_ref, wmean_ref, b_ref,
                        out_ref):
    acc = jnp.dot(pmax_ref[...], wmax_ref[...],
                  preferred_element_type=jnp.float32)
    acc += jnp.dot(pmean_ref[...], wmean_ref[...],
                   preferred_element_type=jnp.float32)
    out_ref[...] = jnp.maximum(acc + b_ref[...], 0.0)


def kernel(obs_encoding, lane_encoding, same_obs_mask, W, b):
    M, D = lane_encoding.shape
    N = obs_encoding.shape[0]
    O = W.shape[0]
    S = (N + NW - 1) // NW
    BN = 512
    NP = ((NW * S + BN - 1) // BN) * BN

    seg = same_obs_mask.reshape(M).astype(jnp.int32)
    bounds = (jnp.arange(NW + 1, dtype=jnp.int32) * S).astype(jnp.int32)
    starts = jnp.searchsorted(seg, bounds, side="left").astype(jnp.int32)
    starts = jnp.zeros((40,), jnp.int32).at[: NW + 1].set(starts)

    pmax, pmean = _seg_pool_kernel(M, D, S, NP)(seg, lane_encoding, starts)

    wmax = W[:, :D].T
    wmean = W[:, D:].T
    b2 = b.reshape(1, O)

    grid = NP // BN
    out = pl.pallas_call(
        _linear_relu_kernel,
        grid=(grid,),
        in_specs=[
            pl.BlockSpec((BN, D), lambda i: (i, 0)),
            pl.BlockSpec((BN, D), lambda i: (i, 0)),
            pl.BlockSpec((D, O), lambda i: (0, 0)),
            pl.BlockSpec((D, O), lambda i: (0, 0)),
            pl.BlockSpec((1, O), lambda i: (0, 0)),
        ],
        out_specs=pl.BlockSpec((BN, O), lambda i: (i, 0)),
        out_shape=jax.ShapeDtypeStruct((NP, O), jnp.float32),
    )(pmax, pmean, wmax, wmean, b2)

    return out[:N]


# speedup vs baseline: 2.2419x; 2.2419x over previous
import functools

import jax
import jax.numpy as jnp
from jax import lax
from jax.experimental import pallas as pl
from jax.experimental.pallas import tpu as pltpu
from jax.experimental.pallas import tpu_sc as plsc

NC = 2
NS = 16
NW = NC * NS
R = 64
DK = 8


def _seg_pool_kernel(M, D, S, NP):
    mesh = plsc.VectorSubcoreMesh(core_axis_name="c", subcore_axis_name="s")
    S1 = S + 1

    @functools.partial(
        pl.kernel,
        out_type=(
            jax.ShapeDtypeStruct((NP, D), jnp.float32),
            jax.ShapeDtypeStruct((NP, D), jnp.float32),
        ),
        mesh=mesh,
        compiler_params=pltpu.CompilerParams(needs_layout_passes=False),
        scratch_types=(
            pltpu.VMEM((48,), jnp.int32),
            pltpu.VMEM((R,), jnp.int32),
            pltpu.VMEM((R, D), jnp.float32),
            pltpu.VMEM((S1, D), jnp.float32),
            pltpu.VMEM((S1, D), jnp.float32),
            pltpu.SMEM((S1,), jnp.int32),
        ),
    )
    def seg_pool(seg_hbm, lanes_hbm, starts_hbm, omax_hbm, omean_hbm,
                 starts_v, seg_buf, rows_buf, acc_max, acc_sum, counts):
        wid = lax.axis_index("s") * NC + lax.axis_index("c")
        base_seg = wid * S

        pltpu.sync_copy(starts_hbm, starts_v)
        iota16 = lax.iota(jnp.int32, 16)
        sv = plsc.load_gather(starts_v, [jnp.minimum(wid + iota16, 47)])
        start = sv[0]
        end = sv[1]
        astart = start - lax.rem(start, 8)
        nt = lax.div(end - astart + (R - 1), R)

        neg_inf = jnp.full((16,), -jnp.inf, dtype=jnp.float32)
        zeros = jnp.zeros((16,), dtype=jnp.float32)

        def init_body(i, _):
            for k in range(DK):
                sl = pl.ds(k * 16, 16)
                acc_max[i, sl] = neg_inf
                acc_sum[i, sl] = zeros
            counts[i] = 0
            return 0

        lax.fori_loop(0, S1, init_body, 0)

        def tile_body(t, _):
            q_t = astart + t * R
            q = pl.multiple_of(jnp.minimum(q_t, M - R), 8)
            pltpu.sync_copy(seg_hbm.at[pl.ds(q, R)], seg_buf)
            pltpu.sync_copy(lanes_hbm.at[pl.ds(q, R), :], rows_buf)
            i_lo = q_t - q
            i_hi = jnp.minimum(end - q, R)

            for g in range(R // 16):
                segv = seg_buf[pl.ds(g * 16, 16)]
                for j in range(16):
                    i = g * 16 + j
                    s = segv[j]
                    valid = jnp.logical_and(i >= i_lo, i < i_hi)
                    loc = s - base_seg
                    loc = jnp.where(loc < 0, S, jnp.minimum(loc, S))
                    loc = jnp.where(valid, loc, S)
                    for k in range(DK):
                        sl = pl.ds(k * 16, 16)
                        r = rows_buf[i, sl]
                        acc_max[loc, sl] = jnp.maximum(acc_max[loc, sl], r)
                        acc_sum[loc, sl] = acc_sum[loc, sl] + r
                    counts[loc] = counts[loc] + 1
            return 0

        lax.fori_loop(0, nt, tile_body, 0)

        def fin_body(i, _):
            c = counts[i]
            cf = jnp.broadcast_to(c, (16,)).astype(jnp.float32)
            inv = 1.0 / jnp.maximum(cf, 1.0)
            nz = cf > 0.0
            for k in range(DK):
                sl = pl.ds(k * 16, 16)
                acc_max[i, sl] = jnp.where(nz, acc_max[i, sl], 0.0)
                acc_sum[i, sl] = acc_sum[i, sl] * inv
            return 0

        lax.fori_loop(0, S, fin_body, 0)

        obase = pl.multiple_of(base_seg, 8)
        pltpu.sync_copy(acc_max.at[pl.ds(0, S), :],
                        omax_hbm.at[pl.ds(obase, S), :])
        pltpu.sync_copy(acc_sum.at[pl.ds(0, S), :],
                        omean_hbm.at[pl.ds(obase, S), :])

    return seg_pool


def _linear_relu_kernel(pmax_ref, pmean_ref, wmax_ref, wmean_ref, b_ref,
                        out_ref):
    acc = jnp.dot(pmax_ref[...], wmax_ref[...],
                  preferred_element_type=jnp.float32)
    acc += jnp.dot(pmean_ref[...], wmean_ref[...],
                   preferred_element_type=jnp.float32)
    out_ref[...] = jnp.maximum(acc + b_ref[...], 0.0)


def kernel(obs_encoding, lane_encoding, same_obs_mask, W, b):
    M, D = lane_encoding.shape
    N = obs_encoding.shape[0]
    O = W.shape[0]
    S = ((N + NW - 1) // NW + 7) // 8 * 8
    BN = 512
    NP = ((NW * S + BN - 1) // BN) * BN

    seg = same_obs_mask.reshape(M).astype(jnp.int32)
    bounds = (jnp.arange(NW + 1, dtype=jnp.int32) * S).astype(jnp.int32)
    starts = jnp.searchsorted(seg, bounds, side="left").astype(jnp.int32)
    starts = jnp.zeros((48,), jnp.int32).at[: NW + 1].set(starts)

    pmax, pmean = _seg_pool_kernel(M, D, S, NP)(seg, lane_encoding, starts)

    wmax = W[:, :D].T
    wmean = W[:, D:].T
    b2 = b.reshape(1, O)

    grid = NP // BN
    out = pl.pallas_call(
        _linear_relu_kernel,
        grid=(grid,),
        in_specs=[
            pl.BlockSpec((BN, D), lambda i: (i, 0)),
            pl.BlockSpec((BN, D), lambda i: (i, 0)),
            pl.BlockSpec((D, O), lambda i: (0, 0)),
            pl.BlockSpec((D, O), lambda i: (0, 0)),
            pl.BlockSpec((1, O), lambda i: (0, 0)),
        ],
        out_specs=pl.BlockSpec((BN, O), lambda i: (i, 0)),
        out_shape=jax.ShapeDtypeStruct((NP, O), jnp.float32),
    )(pmax, pmean, wmax, wmean, b2)

    return out[:N]
